# baseline (speedup 1.0000x reference)
import functools

import jax
import jax.numpy as jnp
from jax import lax
from jax.experimental import pallas as pl
from jax.experimental.pallas import tpu as pltpu
from jax.experimental.pallas import tpu_sc as plsc

F32 = jnp.float32
NC, NS = 2, 16
NW = NC * NS
BLK = 128
DD = 16

N_S, E_S = 2000, 32000
N_L, E_L = 10000, 320000
NP_S, NP_L = 2048, 10112
KS = 8
KL = 80
NB = 4
K0L, K1L = 80, 80

@functools.lru_cache(maxsize=None)
def _mesh():
    return plsc.VectorSubcoreMesh(core_axis_name="c", subcore_axis_name="s")


def _pad_idx(idx, fill, K, B):
    n = NW * K * B
    idx = idx.astype(jnp.int32)
    pad = jnp.full((n - idx.shape[0],), fill, jnp.int32)
    return jnp.concatenate([idx, pad]).reshape(NW, K, B)


def _pad_idx_split(idx, fill, K0, K1, B):
    kmax = max(K0, K1)
    nA, nB = NS * K0 * B, NS * K1 * B
    idx = idx.astype(jnp.int32)
    pad = jnp.full((nA + nB - idx.shape[0],), fill, jnp.int32)
    idxp = jnp.concatenate([idx, pad])
    a = idxp[:nA].reshape(NS, K0, B)
    b = idxp[nA:].reshape(NS, K1, B)
    a = jnp.pad(a, ((0, 0), (0, kmax - K0), (0, 0)), constant_values=fill)
    b = jnp.pad(b, ((0, 0), (0, kmax - K1), (0, 0)), constant_values=fill)
    return jnp.concatenate([a, b], axis=0)


@functools.lru_cache(maxsize=None)
def _sc_gather_multi(specs):
    n_ops = len(specs)
    nbs = [(2 if D >= 256 else NB) if K % NB == 0 else 1 for (_, D, K, _) in specs]
    out_type = [jax.ShapeDtypeStruct((NW * K * B, D), F32) for (_, D, K, B) in specs]
    scratch = [pltpu.VMEM_SHARED((N, D), F32) for (N, D, K, B) in specs]
    for nb, (_, D, K, B) in zip(nbs, specs):
        scratch += [pltpu.VMEM((K, B), jnp.int32)]
        scratch += [pltpu.VMEM((B, D), F32) for _ in range(nb)]
        scratch += [pltpu.SemaphoreType.DMA for _ in range(2 * nb)]

    def body(*refs):
        ins = refs[:2 * n_ops]
        outs = refs[2 * n_ops:3 * n_ops]
        tabs = refs[3 * n_ops:4 * n_ops]
        scr = list(refs[4 * n_ops:])
        cid = lax.axis_index("c")
        sid = lax.axis_index("s")
        wid = cid * NS + sid
        for i, (N, D, K, B) in enumerate(specs):
            st = N // NS
            slt = pl.ds(sid * st, st)
            pltpu.sync_copy(ins[2 * i].at[slt], tabs[i].at[slt])
        plsc.subcore_barrier()
        p = 0
        for i, (nb, (_, D, K, B)) in enumerate(zip(nbs, specs)):
            idx_h = ins[2 * i + 1]
            tab = tabs[i]
            out_h = outs[i]
            idx_v = scr[p]
            rows = scr[p + 1:p + 1 + nb]
            gsem = scr[p + 1 + nb:p + 1 + 2 * nb]
            osem = scr[p + 1 + 2 * nb:p + 1 + 3 * nb]
            p += 1 + 3 * nb
            pltpu.sync_copy(idx_h.at[wid], idx_v)
            G = K // nb
            for b in range(nb):
                pltpu.async_copy(tab.at[idx_v.at[b]], rows[b], gsem[b])

            def outer(g, _, tab=tab, out_h=out_h, idx_v=idx_v,
                      rows=rows, gsem=gsem, osem=osem, nb=nb, G=G, K=K, B=B):
                for b in range(nb):
                    blk = g * nb + b
                    pltpu.make_async_copy(tab.at[idx_v.at[0]], rows[b],
                                          gsem[b]).wait()
                    pltpu.async_copy(rows[b],
                                     out_h.at[pl.ds((wid * K + blk) * B, B)],
                                     osem[b])
                for b in range(nb):
                    pltpu.make_async_copy(rows[b], out_h.at[pl.ds(0, B)],
                                          osem[b]).wait()

                    @pl.when(g + 1 < G)
                    def _issue(b=b, g=g):
                        pltpu.async_copy(tab.at[idx_v.at[(g + 1) * nb + b]],
                                         rows[b], gsem[b])
                return 0

            lax.fori_loop(0, G, outer, 0)

    return pl.kernel(body, out_type=out_type, mesh=_mesh(), scratch_types=scratch,
                     compiler_params=pltpu.CompilerParams(use_tc_tiling_on_sc=False))


@functools.lru_cache(maxsize=None)
def _sc_segsum(D, NP, K0, K1):
    K = max(K0, K1)
    nb = 2
    out_type = jax.ShapeDtypeStruct((NC, NP, D), F32)
    stripe = NP // NS
    scratch = ([pltpu.VMEM((K, BLK), jnp.int32), pltpu.VMEM((K, BLK), jnp.int32),
                pltpu.VMEM_SHARED((NP, D), F32),
                pltpu.VMEM_SHARED((NP, D), F32)]
               + [pltpu.VMEM((BLK, D), F32) for _ in range(nb)]
               + [pltpu.SemaphoreType.DMA for _ in range(2 * nb)])

    def body(table_h, src_h, dst_h, zero_h, out_h, sidx, didx, acc, tab, *ring):
        rows = ring[:nb]
        gsem = ring[nb:2 * nb]
        ssem = ring[2 * nb:3 * nb]
        cid = lax.axis_index("c")
        sid = lax.axis_index("s")
        wid = cid * NS + sid
        sl = pl.ds(sid * stripe, stripe)
        pltpu.sync_copy(zero_h.at[sl], acc.at[sl])
        pltpu.sync_copy(table_h.at[sl], tab.at[sl])
        pltpu.sync_copy(src_h.at[wid], sidx)
        pltpu.sync_copy(dst_h.at[wid], didx)
        plsc.subcore_barrier()

        G = jnp.where(cid == 0, K0 // nb, K1 // nb)

        def outer(g, _):
            for b in range(nb):
                pltpu.async_copy(tab.at[sidx.at[g * nb + b]], rows[b], gsem[b])
            for b in range(nb):
                pltpu.make_async_copy(tab.at[sidx.at[0]], rows[b], gsem[b]).wait()
            for b in range(nb):
                pltpu.async_copy(rows[b], acc.at[didx.at[g * nb + b]], ssem[b], add=True)
            for b in range(nb):
                pltpu.make_async_copy(rows[b], acc.at[didx.at[0]], ssem[b]).wait()
            return 0

        lax.fori_loop(0, G, outer, 0)
        plsc.subcore_barrier()
        pltpu.sync_copy(acc.at[sl], out_h.at[cid, sl])

    return pl.kernel(body, out_type=out_type, mesh=_mesh(), scratch_types=scratch,
                     compiler_params=pltpu.CompilerParams(use_tc_tiling_on_sc=False))


@functools.lru_cache(maxsize=None)
def _sc_scatter_add(D, NP, K):
    out_type = jax.ShapeDtypeStruct((NC, NP, D), F32)
    stripe = NP // NS
    nb = 2 if D >= 256 else NB
    scratch = ([pltpu.VMEM((K, BLK), jnp.int32),
                pltpu.VMEM_SHARED((NP, D), F32)]
               + [pltpu.VMEM((BLK, D), F32) for _ in range(nb)]
               + [pltpu.SemaphoreType.DMA for _ in range(2 * nb)])

    def body(vals_h, dst_h, zero_h, out_h, didx, acc, *ring):
        NB = nb
        rows = ring[:NB]
        gsem = ring[NB:2 * NB]
        ssem = ring[2 * NB:3 * NB]
        cid = lax.axis_index("c")
        sid = lax.axis_index("s")
        wid = cid * NS + sid
        sl = pl.ds(sid * stripe, stripe)
        pltpu.sync_copy(zero_h.at[sl], acc.at[sl])
        pltpu.sync_copy(dst_h.at[wid], didx)
        plsc.subcore_barrier()

        G = K // NB

        def load(blk, b):
            pltpu.async_copy(vals_h.at[pl.ds((wid * K + blk) * BLK, BLK)],
                             rows[b], gsem[b])

        for b in range(NB):
            load(b, b)

        def outer(g, _):
            for b in range(NB):
                blk = g * NB + b
                pltpu.make_async_copy(vals_h.at[pl.ds(0, BLK)], rows[b], gsem[b]).wait()
                pltpu.async_copy(rows[b], acc.at[didx.at[blk]], ssem[b], add=True)
            for b in range(NB):
                pltpu.make_async_copy(rows[b], acc.at[didx.at[0]], ssem[b]).wait()

                @pl.when(g + 1 < G)
                def _issue(b=b, g=g):
                    load((g + 1) * NB + b, b)
            return 0

        lax.fori_loop(0, G, outer, 0)
        plsc.subcore_barrier()
        pltpu.sync_copy(acc.at[sl], out_h.at[cid, sl])

    return pl.kernel(body, out_type=out_type, mesh=_mesh(), scratch_types=scratch,
                     compiler_params=pltpu.CompilerParams(use_tc_tiling_on_sc=False))


@functools.lru_cache(maxsize=None)
def _sc_gat_msg(H, Dh, NP, K, B):
    Do = Dh + 16
    out_type = jax.ShapeDtypeStruct((NC, NP, Do), F32)
    stripe = NP // NS
    scratch = [pltpu.VMEM((K, B), jnp.int32), pltpu.VMEM((K, B), jnp.int32),
               pltpu.VMEM_SHARED((NP, 16), F32), pltpu.VMEM_SHARED((NP, 16), F32),
               pltpu.VMEM_SHARED((NP, Dh), F32),
               pltpu.VMEM_SHARED((NP, Do), F32),
               pltpu.VMEM((B, 16), F32), pltpu.VMEM((B, 16), F32),
               pltpu.VMEM((B, Dh), F32), pltpu.VMEM((B, Do), F32),
               pltpu.SemaphoreType.DMA, pltpu.SemaphoreType.DMA,
               pltpu.SemaphoreType.DMA, pltpu.SemaphoreType.DMA]

    def body(cl_h, cr_h, htab_h, src_h, dst_h, zero_h, out_h,
             sidx, didx, clt, crt, ht, acc, clr, crr, hb, rows, s1, s2, s3, s4):
        cid = lax.axis_index("c")
        sid = lax.axis_index("s")
        wid = cid * NS + sid
        sl = pl.ds(sid * stripe, stripe)
        pltpu.sync_copy(zero_h.at[sl], acc.at[sl])
        pltpu.sync_copy(cl_h.at[sl], clt.at[sl])
        pltpu.sync_copy(cr_h.at[sl], crt.at[sl])
        pltpu.sync_copy(htab_h.at[sl], ht.at[sl])
        pltpu.sync_copy(src_h.at[wid], sidx)
        pltpu.sync_copy(dst_h.at[wid], didx)
        plsc.subcore_barrier()

        def block(j, _):
            pltpu.async_copy(clt.at[sidx.at[j]], clr, s1)
            pltpu.async_copy(crt.at[didx.at[j]], crr, s2)
            pltpu.async_copy(ht.at[sidx.at[j]], hb, s3)
            pltpu.make_async_copy(clt.at[sidx.at[0]], clr, s1).wait()
            pltpu.make_async_copy(crt.at[didx.at[0]], crr, s2).wait()
            pltpu.make_async_copy(ht.at[sidx.at[0]], hb, s3).wait()

            @plsc.parallel_loop(0, B, 1, unroll=8)
            def edge(e):
                x = clr[e] + crr[e]
                w = jnp.exp(jnp.where(x >= 0, x, 0.2 * x))
                rows[e, pl.ds(Dh, 16)] = w
                for h in range(H):
                    sv = jnp.full((16,), w[h], F32)
                    for c in range(4):
                        base = h * 64 + c * 16
                        rows[e, pl.ds(base, 16)] = hb[e, pl.ds(base, 16)] * sv
            pltpu.async_copy(rows, acc.at[didx.at[j]], s4, add=True)
            pltpu.make_async_copy(rows, acc.at[didx.at[0]], s4).wait()
            return 0

        lax.fori_loop(0, K, block, 0)
        plsc.subcore_barrier()
        pltpu.sync_copy(acc.at[sl], out_h.at[cid, sl])

    return pl.kernel(body, out_type=out_type, mesh=_mesh(), scratch_types=scratch,
                     compiler_params=pltpu.CompilerParams(use_tc_tiling_on_sc=False))


@functools.lru_cache(maxsize=None)
def _sc_degree(NP, K):
    out_type = jax.ShapeDtypeStruct((NC, 2, NP, DD), F32)
    stripe = NP // NS
    scratch = ([pltpu.VMEM((K, BLK), jnp.int32), pltpu.VMEM((K, BLK), jnp.int32),
                pltpu.VMEM((BLK, DD), F32),
                pltpu.VMEM_SHARED((NP, DD), F32),
                pltpu.VMEM_SHARED((NP, DD), F32)]
               + [pltpu.SemaphoreType.DMA for _ in range(2 * NB)])

    def body(src_h, dst_h, ones_h, zero_h, out_h, sidx, didx, ones_v,
             acc_s, acc_d, *sems):
        ssem = sems[:NB]
        dsem = sems[NB:2 * NB]
        cid = lax.axis_index("c")
        sid = lax.axis_index("s")
        wid = cid * NS + sid
        sl = pl.ds(sid * stripe, stripe)
        pltpu.sync_copy(zero_h.at[sl], acc_s.at[sl])
        pltpu.sync_copy(zero_h.at[sl], acc_d.at[sl])
        pltpu.sync_copy(ones_h, ones_v)
        pltpu.sync_copy(src_h.at[wid], sidx)
        pltpu.sync_copy(dst_h.at[wid], didx)
        plsc.subcore_barrier()

        G = K // NB

        def outer(g, _):
            for b in range(NB):
                blk = g * NB + b

                @pl.when(g > 0)
                def _drain(b=b):
                    pltpu.make_async_copy(ones_v, acc_s.at[sidx.at[0]], ssem[b]).wait()
                    pltpu.make_async_copy(ones_v, acc_d.at[didx.at[0]], dsem[b]).wait()

                pltpu.async_copy(ones_v, acc_s.at[sidx.at[blk]], ssem[b], add=True)
                pltpu.async_copy(ones_v, acc_d.at[didx.at[blk]], dsem[b], add=True)
            return 0

        lax.fori_loop(0, G, outer, 0)
        for b in range(NB):
            pltpu.make_async_copy(ones_v, acc_s.at[sidx.at[0]], ssem[b]).wait()
            pltpu.make_async_copy(ones_v, acc_d.at[didx.at[0]], dsem[b]).wait()
        plsc.subcore_barrier()
        pltpu.sync_copy(acc_s.at[sl], out_h.at[cid, 0, sl])
        pltpu.sync_copy(acc_d.at[sl], out_h.at[cid, 1, sl])

    return pl.kernel(body, out_type=out_type, mesh=_mesh(), scratch_types=scratch,
                     compiler_params=pltpu.CompilerParams(use_tc_tiling_on_sc=False))


def _leaky(x):
    return jnp.where(x >= 0, x, 0.2 * x)


def _pad_store(ref, val, n):
    ref[pl.ds(0, n), :] = val
    ref[pl.ds(n, ref.shape[0] - n), :] = jnp.zeros(
        (ref.shape[0] - n, ref.shape[1]), F32)


def _tc1a(x, W1, ACL, ACR):
    def body(x_r, w_r, acl_r, acr_r, h_r, cl_r, cr_r):
        h = jnp.dot(x_r[...], w_r[...], preferred_element_type=F32)
        _pad_store(h_r, h, N_S)
        _pad_store(cl_r, jnp.dot(h, acl_r[...], preferred_element_type=F32), N_S)
        _pad_store(cr_r, jnp.dot(h, acr_r[...], preferred_element_type=F32), N_S)

    return pl.pallas_call(
        body,
        out_shape=[jax.ShapeDtypeStruct((NP_S, 256), F32),
                   jax.ShapeDtypeStruct((NP_S, 16), F32),
                   jax.ShapeDtypeStruct((NP_S, 16), F32)],
    )(x, W1, ACL, ACR)


def _tc1b(x, Wg1, degp):
    def body(x_r, w_r, d_r, h_r, din_r, dsrc_r):
        d = d_r[...]
        dout = jnp.maximum(d[:, 0:1] + d[:, 1:2], 1.0)
        din = jnp.maximum(d[:, 2:3] + d[:, 3:4], 1.0)
        dsrc = lax.rsqrt(dout)
        dinv = lax.rsqrt(din)
        h = jnp.dot(x_r[...], w_r[...], preferred_element_type=F32)
        _pad_store(h_r, h * dsrc, N_L)
        din_r[...] = jnp.broadcast_to(dinv, (N_L, 64))
        dsrc_r[...] = jnp.broadcast_to(dsrc, (N_L, 64))

    return pl.pallas_call(
        body,
        out_shape=[jax.ShapeDtypeStruct((NP_L, 64), F32),
                   jax.ShapeDtypeStruct((N_L, 64), F32),
                   jax.ShapeDtypeStruct((N_L, 64), F32)],
    )(x, Wg1, degp)


def _tc2(S1, dinB, dsrcB, bg1, Wg2):
    def body(s_r, di_r, ds_r, bias_r, w_r, o_r):
        s = s_r[0, pl.ds(0, N_L), :] + s_r[1, pl.ds(0, N_L), :]
        g = jax.nn.relu(s * di_r[...] + bias_r[...])
        _pad_store(o_r, jnp.dot(g, w_r[...], preferred_element_type=F32) * ds_r[...],
                   N_L)

    return pl.pallas_call(
        body, out_shape=jax.ShapeDtypeStruct((NP_L, 64), F32),
    )(S1, dinB, dsrcB, bg1, Wg2)


def _tc3(S2, dinB, bg2):
    def body(s_r, di_r, bias_r, o_r):
        s = s_r[0, pl.ds(0, N_L), :] + s_r[1, pl.ds(0, N_L), :]
        _pad_store(o_r, s * di_r[...] + bias_r[...], N_L)

    return pl.pallas_call(
        body, out_shape=jax.ShapeDtypeStruct((NP_L, 64), F32),
    )(S2, dinB, bg2)


def _tc_edge(a_src, a_dst, h_rows, rep, p16, heads):
    Ep, Dh = h_rows.shape
    Do = Dh + 16
    EB = 4096
    grid = (Ep // EB,)

    def body(as_r, ad_r, h_r, rep_r, p_r, o_r):
        w = jnp.exp(_leaky(as_r[:, 0:heads] + ad_r[:, 4:4 + heads]))
        wb = jnp.dot(w, rep_r[...], preferred_element_type=F32)
        wp = jnp.dot(w, p_r[...], preferred_element_type=F32)
        o_r[...] = jnp.concatenate([h_r[...] * wb, wp], axis=1)

    return pl.pallas_call(
        body,
        grid=grid,
        in_specs=[pl.BlockSpec((EB, 16), lambda i: (i, 0)),
                  pl.BlockSpec((EB, 16), lambda i: (i, 0)),
                  pl.BlockSpec((EB, Dh), lambda i: (i, 0)),
                  pl.BlockSpec((heads, Dh), lambda i: (0, 0)),
                  pl.BlockSpec((heads, 16), lambda i: (0, 0))],
        out_specs=pl.BlockSpec((EB, Do), lambda i: (i, 0)),
        out_shape=jax.ShapeDtypeStruct((Ep, Do), F32),
    )(a_src, a_dst, h_rows, rep, p16)


def _tc5(Sm1, rep1, W2, ACL2, ACR2):
    def body(s_r, rep_r, w_r, acl_r, acr_r, h2_r, cl_r, cr_r):
        s = s_r[0, pl.ds(0, N_S), :] + s_r[1, pl.ds(0, N_S), :]
        den = jnp.dot(s[:, 256:260], rep_r[...], preferred_element_type=F32)
        gat1 = jax.nn.relu(s[:, 0:256] / (den + 1e-9))
        h2 = jnp.dot(gat1, w_r[...], preferred_element_type=F32)
        _pad_store(h2_r, h2, N_S)
        _pad_store(cl_r, jnp.dot(h2, acl_r[...], preferred_element_type=F32), N_S)
        _pad_store(cr_r, jnp.dot(h2, acr_r[...], preferred_element_type=F32), N_S)

    return pl.pallas_call(
        body,
        out_shape=[jax.ShapeDtypeStruct((NP_S, 64), F32),
                   jax.ShapeDtypeStruct((NP_S, 16), F32),
                   jax.ShapeDtypeStruct((NP_S, 16), F32)],
    )(Sm1, rep1, W2, ACL2, ACR2)


def _tc7(Sm2, tl, Wc, bc):
    def body(s_r, t_r, w_r, bias_r, o_r):
        s = s_r[0, pl.ds(0, N_S), :] + s_r[1, pl.ds(0, N_S), :]
        gat2 = jax.nn.relu(s[:, 0:64] / (s[:, 64:65] + 1e-9))
        embs = jnp.concatenate([gat2, t_r[pl.ds(0, N_S), :]], axis=1)
        o_r[...] = jnp.dot(embs, w_r[...], preferred_element_type=F32) + bias_r[...]

    return pl.pallas_call(
        body, out_shape=jax.ShapeDtypeStruct((N_S, 32), F32),
    )(Sm2, tl, Wc, bc)


def kernel(small_batch_embs, small_edge_index, token_idx_batch, large_embs,
           large_edge_index, W_gat1, al1, ar1, W_gat2, al2, ar2, Wg1, bg1,
           Wg2, bg2, Wc, bc):
    src_s, dst_s = small_edge_index[0], small_edge_index[1]
    src_l, dst_l = large_edge_index[0], large_edge_index[1]

    src_s_g = _pad_idx(src_s, 0, KS, BLK)
    dst_s_g = _pad_idx(dst_s, 0, KS, BLK)
    dst_s_s = _pad_idx(dst_s, N_S, KS, BLK)
    src_l_g = _pad_idx_split(src_l, 0, K0L, K1L, BLK)
    src_l_s = _pad_idx(src_l, N_L, KL, BLK)
    dst_l_s = _pad_idx(dst_l, N_L, KL, BLK)
    dst_l_sp = _pad_idx_split(dst_l, N_L, K0L, K1L, BLK)
    tok = _pad_idx(token_idx_batch, 0, 1, 64)

    zeros_dd = jnp.zeros((NP_L, DD), F32)
    ones_dd = jnp.ones((BLK, DD), F32)
    zeros_l64 = jnp.zeros((NP_L, 64), F32)
    zeros_s272 = jnp.zeros((NP_S, 272), F32)
    zeros_s80 = jnp.zeros((NP_S, 80), F32)

    degp = _sc_degree(NP_L, KL)(src_l_s, dst_l_s, ones_dd, zeros_dd)
    degp4 = jnp.stack([degp[0, 0, :N_L, 0], degp[1, 0, :N_L, 0],
                       degp[0, 1, :N_L, 0], degp[1, 1, :N_L, 0]], axis=1)

    hL1s, dinB, dsrcB = _tc1b(large_embs, Wg1, degp4)
    S1 = _sc_segsum(64, NP_L, K0L, K1L)(hL1s, src_l_g, dst_l_sp, zeros_l64)
    g1s = _tc2(S1, dinB, dsrcB, bg1.reshape(1, 64), Wg2)
    S2 = _sc_segsum(64, NP_L, K0L, K1L)(g1s, src_l_g, dst_l_sp, zeros_l64)
    g2 = _tc3(S2, dinB, bg2.reshape(1, 64))

    eye4 = jnp.eye(4, dtype=F32)
    AL1 = (eye4[:, None, :] * al1[:, :, None]).reshape(256, 4)
    AR1 = (eye4[:, None, :] * ar1[:, :, None]).reshape(256, 4)
    ACL1 = jnp.concatenate([AL1, jnp.zeros((256, 12), F32)], axis=1)
    ACR1 = jnp.concatenate([AR1, jnp.zeros((256, 12), F32)], axis=1)
    h1, CL1, CR1 = _tc1a(small_batch_embs, W_gat1, ACL1, ACR1)
    REP1 = jnp.repeat(eye4, 64, axis=1)
    src_s_g64 = _pad_idx(src_s, 0, 2 * KS, 64)
    dst_s_s64 = _pad_idx(dst_s, N_S, 2 * KS, 64)
    Sm1 = _sc_gat_msg(4, 256, NP_S, 2 * KS, 64)(CL1, CR1, h1, src_s_g64,
                                                dst_s_s64, zeros_s272)

    ACL2 = jnp.concatenate([al2.reshape(64, 1), jnp.zeros((64, 15), F32)], axis=1)
    ACR2 = jnp.concatenate([ar2.reshape(64, 1), jnp.zeros((64, 15), F32)], axis=1)
    h2, CL2, CR2 = _tc5(Sm1, REP1, W_gat2, ACL2, ACR2)
    Sm2 = _sc_gat_msg(1, 64, NP_S, KS, BLK)(CL2, CR2, h2, src_s_g, dst_s_s,
                                            zeros_s80)

    tl = _sc_gather_multi(((NP_L, 64, 1, 64),))(g2, tok)[0]
    return _tc7(Sm2, tl, Wc, bc.reshape(1, 32))


# speedup vs baseline: 1.0056x; 1.0056x over previous
import functools

import jax
import jax.numpy as jnp
from jax import lax
from jax.experimental import pallas as pl
from jax.experimental.pallas import tpu as pltpu
from jax.experimental.pallas import tpu_sc as plsc

F32 = jnp.float32
NC, NS = 2, 16
NW = NC * NS
BLK = 128
DD = 16

N_S, E_S = 2000, 32000
N_L, E_L = 10000, 320000
NP_S, NP_L = 2048, 10112
KS = 8
KL = 80
NB = 4
K0L, K1L = 80, 80

@functools.lru_cache(maxsize=None)
def _mesh():
    return plsc.VectorSubcoreMesh(core_axis_name="c", subcore_axis_name="s")


def _pad_idx(idx, fill, K, B):
    n = NW * K * B
    idx = idx.astype(jnp.int32)
    pad = jnp.full((n - idx.shape[0],), fill, jnp.int32)
    return jnp.concatenate([idx, pad]).reshape(NW, K, B)


def _pad_idx_split(idx, fill, K0, K1, B):
    kmax = max(K0, K1)
    nA, nB = NS * K0 * B, NS * K1 * B
    idx = idx.astype(jnp.int32)
    pad = jnp.full((nA + nB - idx.shape[0],), fill, jnp.int32)
    idxp = jnp.concatenate([idx, pad])
    a = idxp[:nA].reshape(NS, K0, B)
    b = idxp[nA:].reshape(NS, K1, B)
    a = jnp.pad(a, ((0, 0), (0, kmax - K0), (0, 0)), constant_values=fill)
    b = jnp.pad(b, ((0, 0), (0, kmax - K1), (0, 0)), constant_values=fill)
    return jnp.concatenate([a, b], axis=0)


@functools.lru_cache(maxsize=None)
def _sc_gather_multi(specs):
    n_ops = len(specs)
    nbs = [(2 if D >= 256 else NB) if K % NB == 0 else 1 for (_, D, K, _) in specs]
    out_type = [jax.ShapeDtypeStruct((NW * K * B, D), F32) for (_, D, K, B) in specs]
    scratch = [pltpu.VMEM_SHARED((N, D), F32) for (N, D, K, B) in specs]
    for nb, (_, D, K, B) in zip(nbs, specs):
        scratch += [pltpu.VMEM((K, B), jnp.int32)]
        scratch += [pltpu.VMEM((B, D), F32) for _ in range(nb)]
        scratch += [pltpu.SemaphoreType.DMA for _ in range(2 * nb)]

    def body(*refs):
        ins = refs[:2 * n_ops]
        outs = refs[2 * n_ops:3 * n_ops]
        tabs = refs[3 * n_ops:4 * n_ops]
        scr = list(refs[4 * n_ops:])
        cid = lax.axis_index("c")
        sid = lax.axis_index("s")
        wid = cid * NS + sid
        for i, (N, D, K, B) in enumerate(specs):
            st = N // NS
            slt = pl.ds(sid * st, st)
            pltpu.sync_copy(ins[2 * i].at[slt], tabs[i].at[slt])
        plsc.subcore_barrier()
        p = 0
        for i, (nb, (_, D, K, B)) in enumerate(zip(nbs, specs)):
            idx_h = ins[2 * i + 1]
            tab = tabs[i]
            out_h = outs[i]
            idx_v = scr[p]
            rows = scr[p + 1:p + 1 + nb]
            gsem = scr[p + 1 + nb:p + 1 + 2 * nb]
            osem = scr[p + 1 + 2 * nb:p + 1 + 3 * nb]
            p += 1 + 3 * nb
            pltpu.sync_copy(idx_h.at[wid], idx_v)
            G = K // nb
            for b in range(nb):
                pltpu.async_copy(tab.at[idx_v.at[b]], rows[b], gsem[b])

            def outer(g, _, tab=tab, out_h=out_h, idx_v=idx_v,
                      rows=rows, gsem=gsem, osem=osem, nb=nb, G=G, K=K, B=B):
                for b in range(nb):
                    blk = g * nb + b
                    pltpu.make_async_copy(tab.at[idx_v.at[0]], rows[b],
                                          gsem[b]).wait()
                    pltpu.async_copy(rows[b],
                                     out_h.at[pl.ds((wid * K + blk) * B, B)],
                                     osem[b])
                for b in range(nb):
                    pltpu.make_async_copy(rows[b], out_h.at[pl.ds(0, B)],
                                          osem[b]).wait()

                    @pl.when(g + 1 < G)
                    def _issue(b=b, g=g):
                        pltpu.async_copy(tab.at[idx_v.at[(g + 1) * nb + b]],
                                         rows[b], gsem[b])
                return 0

            lax.fori_loop(0, G, outer, 0)

    return pl.kernel(body, out_type=out_type, mesh=_mesh(), scratch_types=scratch,
                     compiler_params=pltpu.CompilerParams(use_tc_tiling_on_sc=False))


@functools.lru_cache(maxsize=None)
def _sc_segsum(D, NP, K0, K1):
    K = max(K0, K1)
    nb = 2
    out_type = jax.ShapeDtypeStruct((NC, NP, D), F32)
    stripe = NP // NS
    scratch = ([pltpu.VMEM((K, BLK), jnp.int32), pltpu.VMEM((K, BLK), jnp.int32),
                pltpu.VMEM_SHARED((NP, D), F32),
                pltpu.VMEM_SHARED((NP, D), F32)]
               + [pltpu.VMEM((BLK, D), F32) for _ in range(nb)]
               + [pltpu.SemaphoreType.DMA for _ in range(2 * nb)])

    def body(table_h, src_h, dst_h, zero_h, out_h, sidx, didx, acc, tab, *ring):
        rows = ring[:nb]
        gsem = ring[nb:2 * nb]
        ssem = ring[2 * nb:3 * nb]
        cid = lax.axis_index("c")
        sid = lax.axis_index("s")
        wid = cid * NS + sid
        sl = pl.ds(sid * stripe, stripe)
        pltpu.sync_copy(zero_h.at[sl], acc.at[sl])
        pltpu.sync_copy(table_h.at[sl], tab.at[sl])
        pltpu.sync_copy(src_h.at[wid], sidx)
        pltpu.sync_copy(dst_h.at[wid], didx)
        plsc.subcore_barrier()

        G = jnp.where(cid == 0, K0 // nb, K1 // nb)

        def outer(g, _):
            for b in range(nb):
                pltpu.async_copy(tab.at[sidx.at[g * nb + b]], rows[b], gsem[b])
            for b in range(nb):
                pltpu.make_async_copy(tab.at[sidx.at[0]], rows[b], gsem[b]).wait()
            for b in range(nb):
                pltpu.async_copy(rows[b], acc.at[didx.at[g * nb + b]], ssem[b], add=True)
            for b in range(nb):
                pltpu.make_async_copy(rows[b], acc.at[didx.at[0]], ssem[b]).wait()
            return 0

        lax.fori_loop(0, G, outer, 0)
        plsc.subcore_barrier()
        pltpu.sync_copy(acc.at[sl], out_h.at[cid, sl])

    return pl.kernel(body, out_type=out_type, mesh=_mesh(), scratch_types=scratch,
                     compiler_params=pltpu.CompilerParams(use_tc_tiling_on_sc=False))


@functools.lru_cache(maxsize=None)
def _sc_scatter_add(D, NP, K):
    out_type = jax.ShapeDtypeStruct((NC, NP, D), F32)
    stripe = NP // NS
    nb = 2 if D >= 256 else NB
    scratch = ([pltpu.VMEM((K, BLK), jnp.int32),
                pltpu.VMEM_SHARED((NP, D), F32)]
               + [pltpu.VMEM((BLK, D), F32) for _ in range(nb)]
               + [pltpu.SemaphoreType.DMA for _ in range(2 * nb)])

    def body(vals_h, dst_h, zero_h, out_h, didx, acc, *ring):
        NB = nb
        rows = ring[:NB]
        gsem = ring[NB:2 * NB]
        ssem = ring[2 * NB:3 * NB]
        cid = lax.axis_index("c")
        sid = lax.axis_index("s")
        wid = cid * NS + sid
        sl = pl.ds(sid * stripe, stripe)
        pltpu.sync_copy(zero_h.at[sl], acc.at[sl])
        pltpu.sync_copy(dst_h.at[wid], didx)
        plsc.subcore_barrier()

        G = K // NB

        def load(blk, b):
            pltpu.async_copy(vals_h.at[pl.ds((wid * K + blk) * BLK, BLK)],
                             rows[b], gsem[b])

        for b in range(NB):
            load(b, b)

        def outer(g, _):
            for b in range(NB):
                blk = g * NB + b
                pltpu.make_async_copy(vals_h.at[pl.ds(0, BLK)], rows[b], gsem[b]).wait()
                pltpu.async_copy(rows[b], acc.at[didx.at[blk]], ssem[b], add=True)
            for b in range(NB):
                pltpu.make_async_copy(rows[b], acc.at[didx.at[0]], ssem[b]).wait()

                @pl.when(g + 1 < G)
                def _issue(b=b, g=g):
                    load((g + 1) * NB + b, b)
            return 0

        lax.fori_loop(0, G, outer, 0)
        plsc.subcore_barrier()
        pltpu.sync_copy(acc.at[sl], out_h.at[cid, sl])

    return pl.kernel(body, out_type=out_type, mesh=_mesh(), scratch_types=scratch,
                     compiler_params=pltpu.CompilerParams(use_tc_tiling_on_sc=False))


@functools.lru_cache(maxsize=None)
def _sc_gat_msg(H, Dh, NP, K, B):
    Do = Dh + 16
    out_type = jax.ShapeDtypeStruct((NC, NP, Do), F32)
    stripe = NP // NS
    scratch = [pltpu.VMEM((K, B), jnp.int32), pltpu.VMEM((K, B), jnp.int32),
               pltpu.VMEM_SHARED((NP, 16), F32), pltpu.VMEM_SHARED((NP, 16), F32),
               pltpu.VMEM_SHARED((NP, Dh), F32),
               pltpu.VMEM_SHARED((NP, Do), F32),
               pltpu.VMEM((B, 16), F32), pltpu.VMEM((B, 16), F32),
               pltpu.VMEM((B, Dh), F32), pltpu.VMEM((B, Do), F32),
               pltpu.SemaphoreType.DMA, pltpu.SemaphoreType.DMA,
               pltpu.SemaphoreType.DMA, pltpu.SemaphoreType.DMA]

    def body(cl_h, cr_h, htab_h, src_h, dst_h, zero_h, out_h,
             sidx, didx, clt, crt, ht, acc, clr, crr, hb, rows, s1, s2, s3, s4):
        cid = lax.axis_index("c")
        sid = lax.axis_index("s")
        wid = cid * NS + sid
        sl = pl.ds(sid * stripe, stripe)
        pltpu.sync_copy(zero_h.at[sl], acc.at[sl])
        pltpu.sync_copy(cl_h.at[sl], clt.at[sl])
        pltpu.sync_copy(cr_h.at[sl], crt.at[sl])
        pltpu.sync_copy(htab_h.at[sl], ht.at[sl])
        pltpu.sync_copy(src_h.at[wid], sidx)
        pltpu.sync_copy(dst_h.at[wid], didx)
        plsc.subcore_barrier()

        def block(j, _):
            pltpu.async_copy(clt.at[sidx.at[j]], clr, s1)
            pltpu.async_copy(crt.at[didx.at[j]], crr, s2)
            pltpu.async_copy(ht.at[sidx.at[j]], hb, s3)
            pltpu.make_async_copy(clt.at[sidx.at[0]], clr, s1).wait()
            pltpu.make_async_copy(crt.at[didx.at[0]], crr, s2).wait()
            pltpu.make_async_copy(ht.at[sidx.at[0]], hb, s3).wait()

            @plsc.parallel_loop(0, B, 1, unroll=4)
            def edge(e):
                x = clr[e] + crr[e]
                w = jnp.exp(jnp.where(x >= 0, x, 0.2 * x))
                rows[e, pl.ds(Dh, 16)] = w
                for h in range(H):
                    sv = jnp.full((16,), w[h], F32)
                    for c in range(4):
                        base = h * 64 + c * 16
                        rows[e, pl.ds(base, 16)] = hb[e, pl.ds(base, 16)] * sv
            pltpu.async_copy(rows, acc.at[didx.at[j]], s4, add=True)
            pltpu.make_async_copy(rows, acc.at[didx.at[0]], s4).wait()
            return 0

        lax.fori_loop(0, K, block, 0)
        plsc.subcore_barrier()
        pltpu.sync_copy(acc.at[sl], out_h.at[cid, sl])

    return pl.kernel(body, out_type=out_type, mesh=_mesh(), scratch_types=scratch,
                     compiler_params=pltpu.CompilerParams(use_tc_tiling_on_sc=False))


@functools.lru_cache(maxsize=None)
def _sc_degree(NP, K):
    out_type = jax.ShapeDtypeStruct((NC, 2, NP, DD), F32)
    stripe = NP // NS
    scratch = ([pltpu.VMEM((K, BLK), jnp.int32), pltpu.VMEM((K, BLK), jnp.int32),
                pltpu.VMEM((BLK, DD), F32),
                pltpu.VMEM_SHARED((NP, DD), F32),
                pltpu.VMEM_SHARED((NP, DD), F32)]
               + [pltpu.SemaphoreType.DMA for _ in range(2 * NB)])

    def body(src_h, dst_h, ones_h, zero_h, out_h, sidx, didx, ones_v,
             acc_s, acc_d, *sems):
        ssem = sems[:NB]
        dsem = sems[NB:2 * NB]
        cid = lax.axis_index("c")
        sid = lax.axis_index("s")
        wid = cid * NS + sid
        sl = pl.ds(sid * stripe, stripe)
        pltpu.sync_copy(zero_h.at[sl], acc_s.at[sl])
        pltpu.sync_copy(zero_h.at[sl], acc_d.at[sl])
        pltpu.sync_copy(ones_h, ones_v)
        pltpu.sync_copy(src_h.at[wid], sidx)
        pltpu.sync_copy(dst_h.at[wid], didx)
        plsc.subcore_barrier()

        G = K // NB

        def outer(g, _):
            for b in range(NB):
                blk = g * NB + b

                @pl.when(g > 0)
                def _drain(b=b):
                    pltpu.make_async_copy(ones_v, acc_s.at[sidx.at[0]], ssem[b]).wait()
                    pltpu.make_async_copy(ones_v, acc_d.at[didx.at[0]], dsem[b]).wait()

                pltpu.async_copy(ones_v, acc_s.at[sidx.at[blk]], ssem[b], add=True)
                pltpu.async_copy(ones_v, acc_d.at[didx.at[blk]], dsem[b], add=True)
            return 0

        lax.fori_loop(0, G, outer, 0)
        for b in range(NB):
            pltpu.make_async_copy(ones_v, acc_s.at[sidx.at[0]], ssem[b]).wait()
            pltpu.make_async_copy(ones_v, acc_d.at[didx.at[0]], dsem[b]).wait()
        plsc.subcore_barrier()
        pltpu.sync_copy(acc_s.at[sl], out_h.at[cid, 0, sl])
        pltpu.sync_copy(acc_d.at[sl], out_h.at[cid, 1, sl])

    return pl.kernel(body, out_type=out_type, mesh=_mesh(), scratch_types=scratch,
                     compiler_params=pltpu.CompilerParams(use_tc_tiling_on_sc=False))


def _leaky(x):
    return jnp.where(x >= 0, x, 0.2 * x)


def _pad_store(ref, val, n):
    ref[pl.ds(0, n), :] = val
    ref[pl.ds(n, ref.shape[0] - n), :] = jnp.zeros(
        (ref.shape[0] - n, ref.shape[1]), F32)


def _tc1a(x, W1, ACL, ACR):
    def body(x_r, w_r, acl_r, acr_r, h_r, cl_r, cr_r):
        h = jnp.dot(x_r[...], w_r[...], preferred_element_type=F32)
        _pad_store(h_r, h, N_S)
        _pad_store(cl_r, jnp.dot(h, acl_r[...], preferred_element_type=F32), N_S)
        _pad_store(cr_r, jnp.dot(h, acr_r[...], preferred_element_type=F32), N_S)

    return pl.pallas_call(
        body,
        out_shape=[jax.ShapeDtypeStruct((NP_S, 256), F32),
                   jax.ShapeDtypeStruct((NP_S, 16), F32),
                   jax.ShapeDtypeStruct((NP_S, 16), F32)],
    )(x, W1, ACL, ACR)


def _tc1b(x, Wg1, degp):
    def body(x_r, w_r, d_r, h_r, din_r, dsrc_r):
        d = d_r[...]
        dout = jnp.maximum(d[:, 0:1] + d[:, 1:2], 1.0)
        din = jnp.maximum(d[:, 2:3] + d[:, 3:4], 1.0)
        dsrc = lax.rsqrt(dout)
        dinv = lax.rsqrt(din)
        h = jnp.dot(x_r[...], w_r[...], preferred_element_type=F32)
        _pad_store(h_r, h * dsrc, N_L)
        din_r[...] = jnp.broadcast_to(dinv, (N_L, 64))
        dsrc_r[...] = jnp.broadcast_to(dsrc, (N_L, 64))

    return pl.pallas_call(
        body,
        out_shape=[jax.ShapeDtypeStruct((NP_L, 64), F32),
                   jax.ShapeDtypeStruct((N_L, 64), F32),
                   jax.ShapeDtypeStruct((N_L, 64), F32)],
    )(x, Wg1, degp)


def _tc2(S1, dinB, dsrcB, bg1, Wg2):
    def body(s_r, di_r, ds_r, bias_r, w_r, o_r):
        s = s_r[0, pl.ds(0, N_L), :] + s_r[1, pl.ds(0, N_L), :]
        g = jax.nn.relu(s * di_r[...] + bias_r[...])
        _pad_store(o_r, jnp.dot(g, w_r[...], preferred_element_type=F32) * ds_r[...],
                   N_L)

    return pl.pallas_call(
        body, out_shape=jax.ShapeDtypeStruct((NP_L, 64), F32),
    )(S1, dinB, dsrcB, bg1, Wg2)


def _tc3(S2, dinB, bg2):
    def body(s_r, di_r, bias_r, o_r):
        s = s_r[0, pl.ds(0, N_L), :] + s_r[1, pl.ds(0, N_L), :]
        _pad_store(o_r, s * di_r[...] + bias_r[...], N_L)

    return pl.pallas_call(
        body, out_shape=jax.ShapeDtypeStruct((NP_L, 64), F32),
    )(S2, dinB, bg2)


def _tc_edge(a_src, a_dst, h_rows, rep, p16, heads):
    Ep, Dh = h_rows.shape
    Do = Dh + 16
    EB = 4096
    grid = (Ep // EB,)

    def body(as_r, ad_r, h_r, rep_r, p_r, o_r):
        w = jnp.exp(_leaky(as_r[:, 0:heads] + ad_r[:, 4:4 + heads]))
        wb = jnp.dot(w, rep_r[...], preferred_element_type=F32)
        wp = jnp.dot(w, p_r[...], preferred_element_type=F32)
        o_r[...] = jnp.concatenate([h_r[...] * wb, wp], axis=1)

    return pl.pallas_call(
        body,
        grid=grid,
        in_specs=[pl.BlockSpec((EB, 16), lambda i: (i, 0)),
                  pl.BlockSpec((EB, 16), lambda i: (i, 0)),
                  pl.BlockSpec((EB, Dh), lambda i: (i, 0)),
                  pl.BlockSpec((heads, Dh), lambda i: (0, 0)),
                  pl.BlockSpec((heads, 16), lambda i: (0, 0))],
        out_specs=pl.BlockSpec((EB, Do), lambda i: (i, 0)),
        out_shape=jax.ShapeDtypeStruct((Ep, Do), F32),
    )(a_src, a_dst, h_rows, rep, p16)


def _tc5(Sm1, rep1, W2, ACL2, ACR2):
    def body(s_r, rep_r, w_r, acl_r, acr_r, h2_r, cl_r, cr_r):
        s = s_r[0, pl.ds(0, N_S), :] + s_r[1, pl.ds(0, N_S), :]
        den = jnp.dot(s[:, 256:260], rep_r[...], preferred_element_type=F32)
        gat1 = jax.nn.relu(s[:, 0:256] / (den + 1e-9))
        h2 = jnp.dot(gat1, w_r[...], preferred_element_type=F32)
        _pad_store(h2_r, h2, N_S)
        _pad_store(cl_r, jnp.dot(h2, acl_r[...], preferred_element_type=F32), N_S)
        _pad_store(cr_r, jnp.dot(h2, acr_r[...], preferred_element_type=F32), N_S)

    return pl.pallas_call(
        body,
        out_shape=[jax.ShapeDtypeStruct((NP_S, 64), F32),
                   jax.ShapeDtypeStruct((NP_S, 16), F32),
                   jax.ShapeDtypeStruct((NP_S, 16), F32)],
    )(Sm1, rep1, W2, ACL2, ACR2)


def _tc7(Sm2, tl, Wc, bc):
    def body(s_r, t_r, w_r, bias_r, o_r):
        s = s_r[0, pl.ds(0, N_S), :] + s_r[1, pl.ds(0, N_S), :]
        gat2 = jax.nn.relu(s[:, 0:64] / (s[:, 64:65] + 1e-9))
        embs = jnp.concatenate([gat2, t_r[pl.ds(0, N_S), :]], axis=1)
        o_r[...] = jnp.dot(embs, w_r[...], preferred_element_type=F32) + bias_r[...]

    return pl.pallas_call(
        body, out_shape=jax.ShapeDtypeStruct((N_S, 32), F32),
    )(Sm2, tl, Wc, bc)


def kernel(small_batch_embs, small_edge_index, token_idx_batch, large_embs,
           large_edge_index, W_gat1, al1, ar1, W_gat2, al2, ar2, Wg1, bg1,
           Wg2, bg2, Wc, bc):
    src_s, dst_s = small_edge_index[0], small_edge_index[1]
    src_l, dst_l = large_edge_index[0], large_edge_index[1]

    src_s_g = _pad_idx(src_s, 0, KS, BLK)
    dst_s_g = _pad_idx(dst_s, 0, KS, BLK)
    dst_s_s = _pad_idx(dst_s, N_S, KS, BLK)
    src_l_g = _pad_idx_split(src_l, 0, K0L, K1L, BLK)
    src_l_s = _pad_idx(src_l, N_L, KL, BLK)
    dst_l_s = _pad_idx(dst_l, N_L, KL, BLK)
    dst_l_sp = _pad_idx_split(dst_l, N_L, K0L, K1L, BLK)
    tok = _pad_idx(token_idx_batch, 0, 1, 64)

    zeros_dd = jnp.zeros((NP_L, DD), F32)
    ones_dd = jnp.ones((BLK, DD), F32)
    zeros_l64 = jnp.zeros((NP_L, 64), F32)
    zeros_s272 = jnp.zeros((NP_S, 272), F32)
    zeros_s80 = jnp.zeros((NP_S, 80), F32)

    degp = _sc_degree(NP_L, KL)(src_l_s, dst_l_s, ones_dd, zeros_dd)
    degp4 = jnp.stack([degp[0, 0, :N_L, 0], degp[1, 0, :N_L, 0],
                       degp[0, 1, :N_L, 0], degp[1, 1, :N_L, 0]], axis=1)

    hL1s, dinB, dsrcB = _tc1b(large_embs, Wg1, degp4)
    S1 = _sc_segsum(64, NP_L, K0L, K1L)(hL1s, src_l_g, dst_l_sp, zeros_l64)
    g1s = _tc2(S1, dinB, dsrcB, bg1.reshape(1, 64), Wg2)
    S2 = _sc_segsum(64, NP_L, K0L, K1L)(g1s, src_l_g, dst_l_sp, zeros_l64)
    g2 = _tc3(S2, dinB, bg2.reshape(1, 64))

    eye4 = jnp.eye(4, dtype=F32)
    AL1 = (eye4[:, None, :] * al1[:, :, None]).reshape(256, 4)
    AR1 = (eye4[:, None, :] * ar1[:, :, None]).reshape(256, 4)
    ACL1 = jnp.concatenate([AL1, jnp.zeros((256, 12), F32)], axis=1)
    ACR1 = jnp.concatenate([AR1, jnp.zeros((256, 12), F32)], axis=1)
    h1, CL1, CR1 = _tc1a(small_batch_embs, W_gat1, ACL1, ACR1)
    REP1 = jnp.repeat(eye4, 64, axis=1)
    src_s_g64 = _pad_idx(src_s, 0, 2 * KS, 64)
    dst_s_s64 = _pad_idx(dst_s, N_S, 2 * KS, 64)
    Sm1 = _sc_gat_msg(4, 256, NP_S, 2 * KS, 64)(CL1, CR1, h1, src_s_g64,
                                                dst_s_s64, zeros_s272)

    ACL2 = jnp.concatenate([al2.reshape(64, 1), jnp.zeros((64, 15), F32)], axis=1)
    ACR2 = jnp.concatenate([ar2.reshape(64, 1), jnp.zeros((64, 15), F32)], axis=1)
    h2, CL2, CR2 = _tc5(Sm1, REP1, W_gat2, ACL2, ACR2)
    Sm2 = _sc_gat_msg(1, 64, NP_S, KS, BLK)(CL2, CR2, h2, src_s_g, dst_s_s,
                                            zeros_s80)

    tl = _sc_gather_multi(((NP_L, 64, 1, 64),))(g2, tok)[0]
    return _tc7(Sm2, tl, Wc, bc.reshape(1, 32))
